# baseline (speedup 1.0000x reference)
import functools

import jax
import jax.numpy as jnp
from jax import lax
from jax.experimental import pallas as pl
from jax.experimental.pallas import tpu as pltpu
from jax.experimental.pallas import tpu_sc as plsc

N_UNITS = 100000
K_SUB = 5
BATCH = 1024
N_REC = 4096

NC = 2
NS = 16
NW = NC * NS
ROWS_PER_W = BATCH // NW
NBLK = ROWS_PER_W // 8
NIDX = N_REC * K_SUB
LANES = 16
NCHUNKS = N_REC // LANES

SEGW = 1280
NSEG = 78
TAIL0 = NSEG * SEGW
TAILW = 256
NBUCKET = NSEG + 1
NBUF = 5
PADMAX = ((NIDX + NBUCKET * (LANES - 1) + LANES - 1) // LANES) * LANES
DUMP = N_REC
ACC_N = N_REC + LANES
NBND = 96


def _sc_body(e_hbm, et_hbm, pk_hbm, bnd_hbm, cb_hbm, out_hbm,
             buf4, tailbuf, pkv, bndv, cbv, acc2d, or0, or1,
             qsem, semo0, semo1):
    cid = lax.axis_index("c")
    sid = lax.axis_index("s")
    wid = sid * NC + cid
    base = wid * ROWS_PER_W

    pltpu.sync_copy(pk_hbm, pkv)
    pltpu.sync_copy(bnd_hbm, bndv)
    pltpu.sync_copy(cb_hbm.at[pl.ds(base * LANES, ROWS_PER_W * LANES)], cbv)
    pltpu.sync_copy(et_hbm.at[pl.ds(base, ROWS_PER_W)], tailbuf)

    lane = lax.iota(jnp.int32, LANES)
    orows = (or0, or1)
    osems = (semo0, semo1)

    def scalar_at(pos):
        cp = lax.shift_left(lax.shift_right_logical(pos, 4), 4)
        ch = bndv[pl.ds(cp, LANES)]
        return jnp.max(jnp.where(lane == pos - cp, ch, 0))

    def zero_acc():
        def z_body(m, carry):
            acc2d[pl.ds(m * LANES, LANES)] = jnp.zeros((LANES,), jnp.float32)
            return carry
        lax.fori_loop(0, (8 * ACC_N) // LANES, z_body, 0, unroll=4)

    zero_acc()

    def issue(row8, seg):
        pass

    def seg_wait(slot):
        pass

    def extract(lo, hi, gather_fn):
        acc2d[pl.ds(0, LANES)] = (lo + hi) * jnp.ones((LANES,), jnp.float32)

    def blk_body(blk, carry):
        row8 = pl.multiple_of(base + blk * 8, 8)

        for s in range(NBUF):
            issue(row8, s)

        def seg_body(seg, c2):
            slot = lax.rem(seg, NBUF)
            seg_wait(slot)
            lo = scalar_at(seg)
            hi = scalar_at(seg + 1)
            slot_v = jnp.full((LANES,), slot, jnp.int32)

            def g_main(u, addr):
                u_v = jnp.full((LANES,), u, jnp.int32)
                return plsc.load_gather(buf4, [slot_v, u_v, addr])

            extract(lo, hi, g_main)

            @pl.when(seg + NBUF < NSEG)
            def _refill():
                issue(row8, seg + NBUF)
            return c2

        lax.fori_loop(0, NSEG, seg_body, 0)

        t_lo = scalar_at(NSEG)
        t_hi = scalar_at(NSEG + 1)

        def g_tail(u, addr):
            r_v = jnp.full((LANES,), blk * 8 + u, jnp.int32)
            return plsc.load_gather(tailbuf, [r_v, addr])

        extract(t_lo, t_hi, g_tail)

        for u in range(8):
            row_local = blk * 8 + u
            par = u % 2

            @pl.when(blk * 8 + u >= 2)
            def _wait_out():
                pltpu.make_async_copy(
                    orows[par], out_hbm.at[base], osems[par]).wait()

            c16 = cbv[pl.ds(row_local * LANES, LANES)] * jnp.float32(K_SUB)
            orow = orows[par]

            def m_body(m, c2):
                o = m * LANES
                orow[pl.ds(o, LANES)] = c16
                return c2

            lax.fori_loop(0, NCHUNKS, m_body, 0, unroll=2)
            acc2d[pl.ds(u * ACC_N + N_REC, LANES)] = jnp.zeros(
                (LANES,), jnp.float32)

            pltpu.async_copy(orow, out_hbm.at[base + row_local], osems[par])
        return carry

    lax.fori_loop(0, NBLK, blk_body, 0)

    pltpu.make_async_copy(or0, out_hbm.at[base], semo0).wait()
    pltpu.make_async_copy(or1, out_hbm.at[base], semo1).wait()


@jax.jit
def _sc_call(energies, etail, packed, bounds, cb):
    mesh = plsc.VectorSubcoreMesh(core_axis_name="c", subcore_axis_name="s")
    f = functools.partial(
        pl.kernel,
        out_type=jax.ShapeDtypeStruct((BATCH, N_REC), jnp.float32),
        mesh=mesh,
        compiler_params=pltpu.CompilerParams(
            needs_layout_passes=False, use_tc_tiling_on_sc=True),
        scratch_types=[
            pltpu.VMEM((NBUF, 8, SEGW), jnp.float32),
            pltpu.VMEM((ROWS_PER_W, TAILW), jnp.float32),
            pltpu.VMEM((PADMAX,), jnp.int32),
            pltpu.VMEM((NBND,), jnp.int32),
            pltpu.VMEM((ROWS_PER_W * LANES,), jnp.float32),
            pltpu.VMEM((8 * ACC_N,), jnp.float32),
            pltpu.VMEM((N_REC,), jnp.float32),
            pltpu.VMEM((N_REC,), jnp.float32),
            pltpu.SemaphoreType.DMA((NBUF,)),
            pltpu.SemaphoreType.DMA,
            pltpu.SemaphoreType.DMA,
        ],
    )(_sc_body)
    return f(energies, etail, packed, bounds, cb)


def kernel(energies, concentrations, receptor_indices):
    flat = receptor_indices.astype(jnp.int32).T.reshape(-1)
    if True:
        packed0 = jnp.zeros((PADMAX,), jnp.int32)
        bnd0 = jnp.zeros((NBND,), jnp.int32)
        etail0 = jnp.zeros((BATCH, TAILW), jnp.float32)
        cb0 = jnp.zeros((BATCH * LANES,), jnp.float32)
        return _sc_call(energies, etail0, packed0, bnd0, cb0)
    order = jnp.argsort(flat)
    sv = flat[order]
    seg = sv // SEGW
    local = sv - seg * SEGW
    dest = order % N_REC

    counts = jnp.bincount(seg, length=NBUCKET)
    pc = ((counts + LANES - 1) // LANES) * LANES
    pstart = jnp.concatenate([jnp.zeros((1,), jnp.int32),
                              jnp.cumsum(pc)[:-1].astype(jnp.int32)])
    bexcl = (jnp.cumsum(counts) - counts).astype(jnp.int32)
    ppos = pstart[seg] + jnp.arange(NIDX, dtype=jnp.int32) - bexcl[seg]

    pad_dp = DUMP + (jnp.arange(PADMAX, dtype=jnp.int32) % LANES)
    packed = pad_dp.at[ppos].set((local << 16) | dest)

    bnd = jnp.zeros((NBND,), jnp.int32)
    bnd = bnd.at[jnp.arange(NBUCKET)].set(pstart // LANES)
    bnd = bnd.at[NBUCKET].set(
        (pstart[NBUCKET - 1] + pc[NBUCKET - 1]) // LANES)

    etail = jnp.pad(energies[:, TAIL0:], ((0, 0), (0, TAILW - (N_UNITS - TAIL0))))

    cb = jnp.broadcast_to(
        concentrations.reshape(BATCH, 1), (BATCH, LANES)
    ).reshape(-1)
    return _sc_call(etail, packed, bnd, cb)


# speedup vs baseline: 3.8044x; 3.8044x over previous
import functools

import jax
import jax.numpy as jnp
from jax import lax
from jax.experimental import pallas as pl
from jax.experimental.pallas import tpu as pltpu
from jax.experimental.pallas import tpu_sc as plsc

N_UNITS = 100000
K_SUB = 5
BATCH = 1024
N_REC = 4096

NC = 2
NS = 16
NW = NC * NS
R_PER_W = N_REC // NW
LANES = 16
RCHUNK = 8
NSTEP = R_PER_W // RCHUNK
GROWS = RCHUNK * K_SUB
BCHUNKS = BATCH // LANES


def _sc_body(et_hbm, idx_hbm, c_hbm, out_hbm,
             g0, g1, or0, or1, idxv, cv,
             sg0, sg1, so0, so1):
    cid = lax.axis_index("c")
    sid = lax.axis_index("s")
    wid = sid * NC + cid
    ebase = wid * (R_PER_W * K_SUB)
    rbase = wid * R_PER_W

    pltpu.sync_copy(idx_hbm.at[pl.ds(ebase, R_PER_W * K_SUB)], idxv)
    pltpu.sync_copy(c_hbm, cv)

    gbufs = (g0, g1)
    gsems = (sg0, sg1)
    orows = (or0, or1)
    osems = (so0, so1)

    def issue(step, par):
        pltpu.async_copy(
            et_hbm.at[idxv.at[pl.ds(step * GROWS, GROWS)]],
            gbufs[par],
            gsems[par],
        )

    def g_wait(par):
        pltpu.make_async_copy(
            et_hbm.at[pl.ds(0, GROWS)], gbufs[par], gsems[par]).wait()

    issue(0, 0)
    issue(1, 1)

    def step_body(sc, carry):
        for par in range(2):
            step = sc * 2 + par
            g_wait(par)
            gb = gbufs[par]
            orow = orows[par]

            @pl.when(step >= 2)
            def _wait_out():
                pltpu.make_async_copy(
                    orow, out_hbm.at[pl.ds(0, RCHUNK)], osems[par]).wait()

            def m_body(m, c2):
                o = m * LANES
                c5 = cv[pl.ds(o, LANES)] * jnp.float32(K_SUB)
                for rr in range(RCHUNK):
                    s = gb[rr * K_SUB, pl.ds(o, LANES)]
                    for k in range(1, K_SUB):
                        s = s + gb[rr * K_SUB + k, pl.ds(o, LANES)]
                    t = c5 - s
                    p = 1.0 / (1.0 + jnp.exp(-t))
                    orow[rr, pl.ds(o, LANES)] = p
                return c2

            lax.fori_loop(0, BCHUNKS, m_body, 0)

            @pl.when(step + 2 < NSTEP)
            def _refill():
                issue(step + 2, par)

            pltpu.async_copy(
                orow,
                out_hbm.at[pl.ds(rbase + step * RCHUNK, RCHUNK)],
                osems[par],
            )
        return carry

    lax.fori_loop(0, NSTEP // 2, step_body, 0)

    pltpu.make_async_copy(or0, out_hbm.at[pl.ds(0, RCHUNK)], so0).wait()
    pltpu.make_async_copy(or1, out_hbm.at[pl.ds(0, RCHUNK)], so1).wait()


@jax.jit
def _sc_call(et, idxf, conc):
    mesh = plsc.VectorSubcoreMesh(core_axis_name="c", subcore_axis_name="s")
    f = functools.partial(
        pl.kernel,
        out_type=jax.ShapeDtypeStruct((N_REC, BATCH), jnp.float32),
        mesh=mesh,
        compiler_params=pltpu.CompilerParams(
            needs_layout_passes=False, use_tc_tiling_on_sc=True),
        scratch_types=[
            pltpu.VMEM((GROWS, BATCH), jnp.float32),
            pltpu.VMEM((GROWS, BATCH), jnp.float32),
            pltpu.VMEM((RCHUNK, BATCH), jnp.float32),
            pltpu.VMEM((RCHUNK, BATCH), jnp.float32),
            pltpu.VMEM((R_PER_W * K_SUB,), jnp.int32),
            pltpu.VMEM((BATCH,), jnp.float32),
            pltpu.SemaphoreType.DMA,
            pltpu.SemaphoreType.DMA,
            pltpu.SemaphoreType.DMA,
            pltpu.SemaphoreType.DMA,
        ],
    )(_sc_body)
    return f(et, idxf, conc)


def kernel(energies, concentrations, receptor_indices):
    et = energies.T
    idxf = receptor_indices.astype(jnp.int32).reshape(-1)
    out_t = _sc_call(et, idxf, concentrations)
    return out_t.T
